# baseline (speedup 1.0000x reference)
import functools

import jax
import jax.numpy as jnp
import numpy as np
from jax import lax
from jax.experimental import pallas as pl
from jax.experimental.pallas import tpu as pltpu
from jax.experimental.pallas import tpu_sc as plsc

_NS = 8
_L = 16
_CHUNK = 16
_K8E = np.float32(8.0 * np.exp(-1.0))
_MAGIC = np.int32(0x5F3759DF)


def _tables(feats, f_w, f_b, g_w, g_b):
    n, _ = feats.shape
    d = f_w.shape[0]

    def body(x_ref, fwt_ref, gwt_ref, fb_ref, gb_ref, t_ref, u_ref, nf_ref):
        x = x_ref[...]
        tv = jnp.dot(x, fwt_ref[...], preferred_element_type=jnp.float32)
        tv = tv + fb_ref[...]
        uv = jnp.dot(x, gwt_ref[...], preferred_element_type=jnp.float32)
        uv = uv + gb_ref[...]
        t_ref[...] = tv
        u_ref[...] = uv
        nf_ref[...] = jnp.sum(tv * tv, axis=1)

    return pl.pallas_call(
        body,
        out_shape=[
            jax.ShapeDtypeStruct((n, d), jnp.float32),
            jax.ShapeDtypeStruct((n, d), jnp.float32),
            jax.ShapeDtypeStruct((n,), jnp.float32),
        ],
    )(feats, f_w.T, g_w.T, f_b[None, :], g_b[None, :])


def _take16(v, idxv):
    return jnp.take_along_axis(v, idxv, axis=0)


def _sumall(v):
    lane = lax.iota(jnp.int32, _L)
    for sh in (8, 4, 2, 1):
        v = v + _take16(v, lane ^ sh)
    return v


def _dot4(a, b):
    p = a[0] * b[0]
    p = p + a[1] * b[1]
    p = p + a[2] * b[2]
    p = p + a[3] * b[3]
    return p


def _tree_push(stack, v, lane):
    level = 0
    while stack and stack[-1][0] == level:
        _, a = stack.pop()
        d = 1 << level
        m = (lane & d) == 0
        sa = _take16(a, lane ^ d)
        sb = _take16(v, lane ^ d)
        v = jnp.where(m, a, sb) + jnp.where(m, sa, v)
        level += 1
    stack.append((level, v))


_GRP = 4


def _sc_edges(cidx, t_tab, u_tab, nf_tab, n_edges):
    nch = n_edges // _CHUNK
    info = plsc.get_sparse_core_info()
    ncores, nsub = info.num_cores, info.num_subcores
    nw = ncores * nsub
    nch_lo = nch // nw
    rem = nch % nw
    nch_hi = nch_lo + 1
    max_out = nch_hi * _CHUNK
    ngrp_hi = (nch_hi + _GRP - 1) // _GRP
    ng2 = (ngrp_hi + 1) // 2

    mesh = plsc.VectorSubcoreMesh(core_axis_name="c", subcore_axis_name="s")

    @functools.partial(
        pl.kernel,
        out_type=jax.ShapeDtypeStruct((n_edges,), jnp.float32),
        mesh=mesh,
        compiler_params=pltpu.CompilerParams(
            needs_layout_passes=False, use_tc_tiling_on_sc=False),
        scratch_types=[
            pltpu.VMEM((_GRP, 320), jnp.int32),
            pltpu.VMEM((_GRP, 320), jnp.int32),
            pltpu.VMEM((32, 64), jnp.float32),
            pltpu.VMEM((32, 64), jnp.float32),
            pltpu.VMEM((32, 64), jnp.float32),
            pltpu.VMEM((32, 64), jnp.float32),
            pltpu.VMEM((256, 64), jnp.float32),
            pltpu.VMEM((256, 64), jnp.float32),
            pltpu.VMEM((10000,), jnp.float32),
            pltpu.VMEM((max_out,), jnp.float32),
            pltpu.SemaphoreType.DMA,
            pltpu.SemaphoreType.DMA,
            pltpu.SemaphoreType.DMA,
            pltpu.SemaphoreType.DMA,
        ],
    )
    def k(cidx_hbm, t_hbm, u_hbm, nf_hbm, out_hbm,
          cb0, cb1, tsd0, tsd1, gds0, gds1, mb0, mb1, nf_v, out_v,
          si0, si1, sg0, sg1):
        wid = lax.axis_index("c") * nsub + lax.axis_index("s")
        is_hi = wid < rem
        base = jnp.where(is_hi, wid * nch_hi, wid * nch_lo + rem)
        nch_t = jnp.where(is_hi, nch_hi, nch_lo)

        cbs = (cb0, cb1)
        tsds = (tsd0, tsd1)
        gdss = (gds0, gds1)
        mbs = (mb0, mb1)
        sis = (si0, si1)
        sgs = (sg0, sg1)

        pltpu.sync_copy(nf_hbm, nf_v)

        lane = lax.iota(jnp.int32, _L)
        mask8 = lane < _NS
        lane0 = lane == 0

        def idx_desc(bp, g):
            return pltpu.make_async_copy(
                cidx_hbm.at[pl.ds(base + g * _GRP, _GRP), :], cbs[bp], sis[bp])

        def gather_descs(sp, bp, j):
            cb = cbs[bp]
            return (
                pltpu.make_async_copy(t_hbm.at[cb.at[j, pl.ds(0, 32)]],
                                      tsds[sp], sgs[sp]),
                pltpu.make_async_copy(u_hbm.at[cb.at[j, pl.ds(32, 32)]],
                                      gdss[sp], sgs[sp]),
                pltpu.make_async_copy(t_hbm.at[cb.at[j, pl.ds(64, 128)]],
                                      mbs[sp].at[pl.ds(0, 128)], sgs[sp]),
                pltpu.make_async_copy(t_hbm.at[cb.at[j, pl.ds(192, 128)]],
                                      mbs[sp].at[pl.ds(128, 128)], sgs[sp]),
            )

        def compute_chunk(c, sp, bp, j):
            tsd_v, gds_v, mb_v, cb = tsds[sp], gdss[sp], mbs[sp], cbs[bp]
            sv = cb[j, pl.ds(0, _L)]
            dv = cb[j, pl.ds(_L, _L)]

            def edge_compute(e):
                fs = tuple(tsd_v[e, pl.ds(16 * q, 16)] for q in range(4))
                fd = tuple(tsd_v[16 + e, pl.ds(16 * q, 16)] for q in range(4))
                gd = tuple(gds_v[e, pl.ds(16 * q, 16)] for q in range(4))
                gs = tuple(gds_v[16 + e, pl.ds(16 * q, 16)] for q in range(4))
                s1v = _sumall(_dot4(fs, gd))
                s2v = _sumall(_dot4(fd, gs))
                cstack, astack = [], []
                for s in range(16):
                    m = tuple(mb_v[e * 16 + s, pl.ds(16 * q, 16)]
                              for q in range(4))
                    fo, go = (fs, gd) if s < _NS else (fd, gs)
                    _tree_push(cstack, _dot4(m, fo), lane)
                    _tree_push(astack, _dot4(m, go), lane)
                cvec = cstack[0][1]
                avec = astack[0][1]
                nmidx = cb[j, pl.ds(64 + e * 16, 16)]
                nmv = plsc.load_gather(nf_v, [nmidx])
                egv = jnp.full((_L,), e, jnp.int32)
                sev = _take16(sv, egv)
                dev = _take16(dv, egv)
                nidx = jnp.where(mask8, sev, dev)
                nownv = plsc.load_gather(nf_v, [nidx])
                qv = jnp.maximum(nownv + nmv - 2.0 * cvec, 0.0)
                qc = jnp.maximum(qv, 1e-30)
                ii = _MAGIC - (plsc.bitcast(qc, jnp.int32) >> 1)
                y = plsc.bitcast(ii, jnp.float32)
                for _ in range(4):
                    y = y * (1.5 - 0.5 * qc * y * y)
                dvv = qv * y
                ev = jnp.exp(-dvv)
                svv = jnp.where(mask8, s1v, s2v)
                numt = _sumall(ev * (svv - avec))
                zv = _sumall(ev) + _K8E
                r = numt / zv
                outv = 1.0 / (1.0 + jnp.exp(-r))
                pos = jnp.full((_L,), c * _CHUNK + e, jnp.int32)
                plsc.store_scatter(out_v, [pos], outv, mask=lane0)

            def ebody2(h, ecarry):
                edge_compute(2 * h)
                edge_compute(2 * h + 1)
                return ecarry

            lax.fori_loop(0, 8, ebody2, 0)

        d = idx_desc(0, 0)
        d.start()
        d.wait()
        for g in gather_descs(0, 0, 0):
            g.start()

        def g2_body(g2, carry):
            for gp in (0, 1):
                g = 2 * g2 + gp
                for j in range(_GRP):
                    c = g * _GRP + j
                    sp = j % 2
                    if j == 0:
                        @pl.when((g + 1) * _GRP < nch_t)
                        def _(g=g, gp=gp):
                            idx_desc(1 - gp, g + 1).start()
                    if j == _GRP - 1:
                        @pl.when(c + 1 < nch_t)
                        def _(g=g, gp=gp, sp=sp):
                            idx_desc(1 - gp, g + 1).wait()
                            for gg in gather_descs(1 - sp, 1 - gp, 0):
                                gg.start()
                    else:
                        @pl.when(c + 1 < nch_t)
                        def _(gp=gp, sp=sp, j=j):
                            for gg in gather_descs(1 - sp, gp, j + 1):
                                gg.start()

                    @pl.when(c < nch_t)
                    def _(c=c, sp=sp, gp=gp, j=j):
                        for gg in gather_descs(sp, gp, j):
                            gg.wait()
                        compute_chunk(c, sp, gp, j)
            return carry

        lax.fori_loop(0, ng2, g2_body, 0)

        ebase = base * _CHUNK

        @pl.when(is_hi)
        def _():
            pltpu.sync_copy(out_v,
                            out_hbm.at[pl.ds(ebase, nch_hi * _CHUNK)])

        @pl.when(jnp.logical_not(is_hi))
        def _():
            pltpu.sync_copy(out_v.at[pl.ds(0, nch_lo * _CHUNK)],
                            out_hbm.at[pl.ds(ebase, nch_lo * _CHUNK)])

    return k(cidx, t_tab, u_tab, nf_tab)


def kernel(src, dst, feats, f_w, f_b, g_w, g_b):
    n_edges = src.shape[0]
    n_nodes = feats.shape[0]
    src = src.astype(jnp.int32)
    dst = dst.astype(jnp.int32)

    rk = jax.random.key(42)
    mid0 = jax.random.randint(jax.random.fold_in(rk, 0), (n_edges, _NS), 0,
                              n_nodes).astype(jnp.int32)
    mid1 = jax.random.randint(jax.random.fold_in(rk, 1), (n_edges, _NS), 0,
                              n_nodes).astype(jnp.int32)

    t_tab, u_tab, nf_tab = _tables(feats, f_w, f_b, g_w, g_b)

    nch = n_edges // _CHUNK
    srcr = src.reshape(nch, _CHUNK)
    dstr = dst.reshape(nch, _CHUNK)
    m0 = mid0.reshape(nch, _CHUNK, _NS)
    m1 = mid1.reshape(nch, _CHUNK, _NS)
    midc = jnp.concatenate([m0, m1], axis=2).reshape(nch, 256)
    cidx = jnp.concatenate([srcr, dstr, dstr, srcr, midc], axis=1)

    return _sc_edges(cidx, t_tab, u_tab, nf_tab, n_edges)


# speedup vs baseline: 1.1286x; 1.1286x over previous
import functools

import jax
import jax.numpy as jnp
import numpy as np
from jax import lax
from jax.experimental import pallas as pl
from jax.experimental.pallas import tpu as pltpu
from jax.experimental.pallas import tpu_sc as plsc

_NS = 8
_L = 16
_CHUNK = 16
_K8E = np.float32(8.0 * np.exp(-1.0))
_MAGIC = np.int32(0x5F3759DF)


def _tables(feats, f_w, f_b, g_w, g_b):
    n, _ = feats.shape
    d = f_w.shape[0]

    def body(x_ref, fwt_ref, gwt_ref, fb_ref, gb_ref, t_ref, u_ref, nf_ref):
        x = x_ref[...]
        tv = jnp.dot(x, fwt_ref[...], preferred_element_type=jnp.float32)
        tv = tv + fb_ref[...]
        uv = jnp.dot(x, gwt_ref[...], preferred_element_type=jnp.float32)
        uv = uv + gb_ref[...]
        t_ref[...] = tv
        u_ref[...] = uv
        nf_ref[...] = jnp.sum(tv * tv, axis=1)

    return pl.pallas_call(
        body,
        out_shape=[
            jax.ShapeDtypeStruct((n, d), jnp.float32),
            jax.ShapeDtypeStruct((n, d), jnp.float32),
            jax.ShapeDtypeStruct((n,), jnp.float32),
        ],
    )(feats, f_w.T, g_w.T, f_b[None, :], g_b[None, :])


def _take16(v, idxv):
    return jnp.take_along_axis(v, idxv, axis=0)


def _sumall(v):
    lane = lax.iota(jnp.int32, _L)
    for sh in (8, 4, 2, 1):
        v = v + _take16(v, lane ^ sh)
    return v


def _dot4(a, b):
    p = a[0] * b[0]
    p = p + a[1] * b[1]
    p = p + a[2] * b[2]
    p = p + a[3] * b[3]
    return p


def _tree_push(stack, v, lane):
    level = 0
    while stack and stack[-1][0] == level:
        _, a = stack.pop()
        d = 1 << level
        m = (lane & d) == 0
        sa = _take16(a, lane ^ d)
        sb = _take16(v, lane ^ d)
        v = jnp.where(m, a, sb) + jnp.where(m, sa, v)
        level += 1
    stack.append((level, v))


_GRP = 2


def _sc_edges(cidx, t_tab, u_tab, nf_tab, n_edges):
    nch = n_edges // _CHUNK
    info = plsc.get_sparse_core_info()
    ncores, nsub = info.num_cores, info.num_subcores
    nw = ncores * nsub
    nch_lo = nch // nw
    rem = nch % nw
    nch_hi = nch_lo + 1
    max_out = nch_hi * _CHUNK
    ngrp_hi = (nch_hi + _GRP - 1) // _GRP
    ng2 = (ngrp_hi + 1) // 2

    mesh = plsc.VectorSubcoreMesh(core_axis_name="c", subcore_axis_name="s")

    @functools.partial(
        pl.kernel,
        out_type=jax.ShapeDtypeStruct((n_edges,), jnp.float32),
        mesh=mesh,
        compiler_params=pltpu.CompilerParams(
            needs_layout_passes=False, use_tc_tiling_on_sc=False),
        scratch_types=[
            pltpu.VMEM((_GRP, 320), jnp.int32),
            pltpu.VMEM((_GRP, 320), jnp.int32),
            pltpu.VMEM((32, 64), jnp.float32),
            pltpu.VMEM((32, 64), jnp.float32),
            pltpu.VMEM((32, 64), jnp.float32),
            pltpu.VMEM((32, 64), jnp.float32),
            pltpu.VMEM((256, 64), jnp.float32),
            pltpu.VMEM((256, 64), jnp.float32),
            pltpu.VMEM((10000,), jnp.float32),
            pltpu.VMEM((max_out,), jnp.float32),
            pltpu.SemaphoreType.DMA,
            pltpu.SemaphoreType.DMA,
            pltpu.SemaphoreType.DMA,
            pltpu.SemaphoreType.DMA,
        ],
    )
    def k(cidx_hbm, t_hbm, u_hbm, nf_hbm, out_hbm,
          cb0, cb1, tsd0, tsd1, gds0, gds1, mb0, mb1, nf_v, out_v,
          si0, si1, sg0, sg1):
        wid = lax.axis_index("c") * nsub + lax.axis_index("s")
        is_hi = wid < rem
        base = jnp.where(is_hi, wid * nch_hi, wid * nch_lo + rem)
        nch_t = jnp.where(is_hi, nch_hi, nch_lo)

        cbs = (cb0, cb1)
        tsds = (tsd0, tsd1)
        gdss = (gds0, gds1)
        mbs = (mb0, mb1)
        sis = (si0, si1)
        sgs = (sg0, sg1)

        pltpu.sync_copy(nf_hbm, nf_v)

        lane = lax.iota(jnp.int32, _L)
        mask8 = lane < _NS
        lane0 = lane == 0

        def idx_desc(bp, g):
            return pltpu.make_async_copy(
                cidx_hbm.at[pl.ds(base + g * _GRP, _GRP), :], cbs[bp], sis[bp])

        def gather_descs(sp, bp, j):
            cb = cbs[bp]
            return (
                pltpu.make_async_copy(t_hbm.at[cb.at[j, pl.ds(0, 32)]],
                                      tsds[sp], sgs[sp]),
                pltpu.make_async_copy(u_hbm.at[cb.at[j, pl.ds(32, 32)]],
                                      gdss[sp], sgs[sp]),
                pltpu.make_async_copy(t_hbm.at[cb.at[j, pl.ds(64, 128)]],
                                      mbs[sp].at[pl.ds(0, 128)], sgs[sp]),
                pltpu.make_async_copy(t_hbm.at[cb.at[j, pl.ds(192, 128)]],
                                      mbs[sp].at[pl.ds(128, 128)], sgs[sp]),
            )

        def compute_chunk(c, sp, bp, j):
            tsd_v, gds_v, mb_v, cb = tsds[sp], gdss[sp], mbs[sp], cbs[bp]
            sv = cb[j, pl.ds(0, _L)]
            dv = cb[j, pl.ds(_L, _L)]

            def edge_compute(e):
                fs = tuple(tsd_v[e, pl.ds(16 * q, 16)] for q in range(4))
                fd = tuple(tsd_v[16 + e, pl.ds(16 * q, 16)] for q in range(4))
                gd = tuple(gds_v[e, pl.ds(16 * q, 16)] for q in range(4))
                gs = tuple(gds_v[16 + e, pl.ds(16 * q, 16)] for q in range(4))
                s1v = _sumall(_dot4(fs, gd))
                s2v = _sumall(_dot4(fd, gs))
                cstack, astack = [], []
                for s in range(16):
                    m = tuple(mb_v[e * 16 + s, pl.ds(16 * q, 16)]
                              for q in range(4))
                    fo, go = (fs, gd) if s < _NS else (fd, gs)
                    _tree_push(cstack, _dot4(m, fo), lane)
                    _tree_push(astack, _dot4(m, go), lane)
                cvec = cstack[0][1]
                avec = astack[0][1]
                nmidx = cb[j, pl.ds(64 + e * 16, 16)]
                nmv = plsc.load_gather(nf_v, [nmidx])
                egv = jnp.full((_L,), e, jnp.int32)
                sev = _take16(sv, egv)
                dev = _take16(dv, egv)
                nidx = jnp.where(mask8, sev, dev)
                nownv = plsc.load_gather(nf_v, [nidx])
                qv = jnp.maximum(nownv + nmv - 2.0 * cvec, 0.0)
                qc = jnp.maximum(qv, 1e-30)
                ii = _MAGIC - (plsc.bitcast(qc, jnp.int32) >> 1)
                y = plsc.bitcast(ii, jnp.float32)
                for _ in range(4):
                    y = y * (1.5 - 0.5 * qc * y * y)
                dvv = qv * y
                ev = jnp.exp(-dvv)
                svv = jnp.where(mask8, s1v, s2v)
                numt = _sumall(ev * (svv - avec))
                zv = _sumall(ev) + _K8E
                r = numt / zv
                outv = 1.0 / (1.0 + jnp.exp(-r))
                pos = jnp.full((_L,), c * _CHUNK + e, jnp.int32)
                plsc.store_scatter(out_v, [pos], outv, mask=lane0)

            def ebody(h, ecarry):
                edge_compute(h)
                return ecarry

            lax.fori_loop(0, 16, ebody, 0)

        d = idx_desc(0, 0)
        d.start()
        d.wait()
        for g in gather_descs(0, 0, 0):
            g.start()

        def g2_body(g2, carry):
            for gp in (0, 1):
                g = 2 * g2 + gp
                for j in range(_GRP):
                    c = g * _GRP + j
                    sp = j % 2
                    if j == 0:
                        @pl.when((g + 1) * _GRP < nch_t)
                        def _(g=g, gp=gp):
                            idx_desc(1 - gp, g + 1).start()
                    if j == _GRP - 1:
                        @pl.when(c + 1 < nch_t)
                        def _(g=g, gp=gp, sp=sp):
                            idx_desc(1 - gp, g + 1).wait()
                            for gg in gather_descs(1 - sp, 1 - gp, 0):
                                gg.start()
                    else:
                        @pl.when(c + 1 < nch_t)
                        def _(gp=gp, sp=sp, j=j):
                            for gg in gather_descs(1 - sp, gp, j + 1):
                                gg.start()

                    @pl.when(c < nch_t)
                    def _(c=c, sp=sp, gp=gp, j=j):
                        for gg in gather_descs(sp, gp, j):
                            gg.wait()
                        compute_chunk(c, sp, gp, j)
            return carry

        lax.fori_loop(0, ng2, g2_body, 0)

        ebase = base * _CHUNK

        @pl.when(is_hi)
        def _():
            pltpu.sync_copy(out_v,
                            out_hbm.at[pl.ds(ebase, nch_hi * _CHUNK)])

        @pl.when(jnp.logical_not(is_hi))
        def _():
            pltpu.sync_copy(out_v.at[pl.ds(0, nch_lo * _CHUNK)],
                            out_hbm.at[pl.ds(ebase, nch_lo * _CHUNK)])

    return k(cidx, t_tab, u_tab, nf_tab)


def kernel(src, dst, feats, f_w, f_b, g_w, g_b):
    n_edges = src.shape[0]
    n_nodes = feats.shape[0]
    src = src.astype(jnp.int32)
    dst = dst.astype(jnp.int32)

    rk = jax.random.key(42)
    mid0 = jax.random.randint(jax.random.fold_in(rk, 0), (n_edges, _NS), 0,
                              n_nodes).astype(jnp.int32)
    mid1 = jax.random.randint(jax.random.fold_in(rk, 1), (n_edges, _NS), 0,
                              n_nodes).astype(jnp.int32)

    t_tab, u_tab, nf_tab = _tables(feats, f_w, f_b, g_w, g_b)

    nch = n_edges // _CHUNK
    srcr = src.reshape(nch, _CHUNK)
    dstr = dst.reshape(nch, _CHUNK)
    m0 = mid0.reshape(nch, _CHUNK, _NS)
    m1 = mid1.reshape(nch, _CHUNK, _NS)
    midc = jnp.concatenate([m0, m1], axis=2).reshape(nch, 256)
    cidx = jnp.concatenate([srcr, dstr, dstr, srcr, midc], axis=1)

    return _sc_edges(cidx, t_tab, u_tab, nf_tab, n_edges)


# speedup vs baseline: 1.1378x; 1.0082x over previous
import functools

import jax
import jax.numpy as jnp
import numpy as np
from jax import lax
from jax.experimental import pallas as pl
from jax.experimental.pallas import tpu as pltpu
from jax.experimental.pallas import tpu_sc as plsc

_NS = 8
_L = 16
_CHUNK = 16
_K8E = np.float32(8.0 * np.exp(-1.0))
_MAGIC = np.int32(0x5F3759DF)


def _tables(feats, f_w, f_b, g_w, g_b):
    n, _ = feats.shape
    d = f_w.shape[0]

    def body(x_ref, fwt_ref, gwt_ref, fb_ref, gb_ref, t_ref, u_ref, nf_ref):
        x = x_ref[...]
        tv = jnp.dot(x, fwt_ref[...], preferred_element_type=jnp.float32)
        tv = tv + fb_ref[...]
        uv = jnp.dot(x, gwt_ref[...], preferred_element_type=jnp.float32)
        uv = uv + gb_ref[...]
        t_ref[...] = tv
        u_ref[...] = uv
        nf_ref[...] = jnp.sum(tv * tv, axis=1)

    return pl.pallas_call(
        body,
        out_shape=[
            jax.ShapeDtypeStruct((n, d), jnp.float32),
            jax.ShapeDtypeStruct((n, d), jnp.float32),
            jax.ShapeDtypeStruct((n,), jnp.float32),
        ],
    )(feats, f_w.T, g_w.T, f_b[None, :], g_b[None, :])


def _take16(v, idxv):
    return jnp.take_along_axis(v, idxv, axis=0)


def _sumall(v):
    lane = lax.iota(jnp.int32, _L)
    for sh in (8, 4, 2, 1):
        v = v + _take16(v, lane ^ sh)
    return v


def _dot4(a, b):
    p = a[0] * b[0]
    p = p + a[1] * b[1]
    p = p + a[2] * b[2]
    p = p + a[3] * b[3]
    return p


def _tree_push(stack, v, lane):
    level = 0
    while stack and stack[-1][0] == level:
        _, a = stack.pop()
        d = 1 << level
        m = (lane & d) == 0
        sa = _take16(a, lane ^ d)
        sb = _take16(v, lane ^ d)
        v = jnp.where(m, a, sb) + jnp.where(m, sa, v)
        level += 1
    stack.append((level, v))


_GRP = 2


def _sc_edges(cidx, t_tab, u_tab, nf_tab, n_edges):
    nch = n_edges // _CHUNK
    info = plsc.get_sparse_core_info()
    ncores, nsub = info.num_cores, info.num_subcores
    nw = ncores * nsub
    nch_lo = nch // nw
    rem = nch % nw
    nch_hi = nch_lo + 1
    max_out = nch_hi * _CHUNK
    ngrp_hi = (nch_hi + _GRP - 1) // _GRP
    ng2 = (ngrp_hi + 1) // 2

    mesh = plsc.VectorSubcoreMesh(core_axis_name="c", subcore_axis_name="s")

    @functools.partial(
        pl.kernel,
        out_type=jax.ShapeDtypeStruct((n_edges,), jnp.float32),
        mesh=mesh,
        compiler_params=pltpu.CompilerParams(
            needs_layout_passes=False, use_tc_tiling_on_sc=False),
        scratch_types=[
            pltpu.VMEM((_GRP, 320), jnp.int32),
            pltpu.VMEM((_GRP, 320), jnp.int32),
            pltpu.VMEM((32, 64), jnp.float32),
            pltpu.VMEM((32, 64), jnp.float32),
            pltpu.VMEM((32, 64), jnp.float32),
            pltpu.VMEM((32, 64), jnp.float32),
            pltpu.VMEM((256, 64), jnp.float32),
            pltpu.VMEM((256, 64), jnp.float32),
            pltpu.VMEM((10000,), jnp.float32),
            pltpu.VMEM((max_out,), jnp.float32),
            pltpu.SemaphoreType.DMA,
            pltpu.SemaphoreType.DMA,
            pltpu.SemaphoreType.DMA,
            pltpu.SemaphoreType.DMA,
        ],
    )
    def k(cidx_hbm, t_hbm, u_hbm, nf_hbm, out_hbm,
          cb0, cb1, tsd0, tsd1, gds0, gds1, mb0, mb1, nf_v, out_v,
          si0, si1, sg0, sg1):
        wid = lax.axis_index("c") * nsub + lax.axis_index("s")
        is_hi = wid < rem
        base = jnp.where(is_hi, wid * nch_hi, wid * nch_lo + rem)
        nch_t = jnp.where(is_hi, nch_hi, nch_lo)

        cbs = (cb0, cb1)
        tsds = (tsd0, tsd1)
        gdss = (gds0, gds1)
        mbs = (mb0, mb1)
        sis = (si0, si1)
        sgs = (sg0, sg1)

        pltpu.sync_copy(nf_hbm, nf_v)

        lane = lax.iota(jnp.int32, _L)
        mask8 = lane < _NS
        lane0 = lane == 0

        def idx_desc(bp, g):
            return pltpu.make_async_copy(
                cidx_hbm.at[pl.ds(base + g * _GRP, _GRP), :], cbs[bp], sis[bp])

        def gather_descs(sp, bp, j):
            cb = cbs[bp]
            return (
                pltpu.make_async_copy(t_hbm.at[cb.at[j, pl.ds(0, 32)]],
                                      tsds[sp], sgs[sp]),
                pltpu.make_async_copy(u_hbm.at[cb.at[j, pl.ds(32, 32)]],
                                      gdss[sp], sgs[sp]),
                pltpu.make_async_copy(t_hbm.at[cb.at[j, pl.ds(64, 128)]],
                                      mbs[sp].at[pl.ds(0, 128)], sgs[sp]),
                pltpu.make_async_copy(t_hbm.at[cb.at[j, pl.ds(192, 128)]],
                                      mbs[sp].at[pl.ds(128, 128)], sgs[sp]),
            )

        def compute_chunk(c, sp, bp, j):
            tsd_v, gds_v, mb_v, cb = tsds[sp], gdss[sp], mbs[sp], cbs[bp]
            sv = cb[j, pl.ds(0, _L)]
            dv = cb[j, pl.ds(_L, _L)]

            def edge_compute(e):
                fs = tuple(tsd_v[e, pl.ds(16 * q, 16)] for q in range(4))
                fd = tuple(tsd_v[16 + e, pl.ds(16 * q, 16)] for q in range(4))
                gd = tuple(gds_v[e, pl.ds(16 * q, 16)] for q in range(4))
                gs = tuple(gds_v[16 + e, pl.ds(16 * q, 16)] for q in range(4))
                s1v = _sumall(_dot4(fs, gd))
                s2v = _sumall(_dot4(fd, gs))
                cstack, astack = [], []
                for s in range(16):
                    m = tuple(mb_v[e * 16 + s, pl.ds(16 * q, 16)]
                              for q in range(4))
                    fo, go = (fs, gd) if s < _NS else (fd, gs)
                    _tree_push(cstack, _dot4(m, fo), lane)
                    _tree_push(astack, _dot4(m, go), lane)
                cvec = cstack[0][1]
                avec = astack[0][1]
                nmidx = cb[j, pl.ds(64 + e * 16, 16)]
                nmv = plsc.load_gather(nf_v, [nmidx])
                egv = jnp.full((_L,), e, jnp.int32)
                sev = _take16(sv, egv)
                dev = _take16(dv, egv)
                nidx = jnp.where(mask8, sev, dev)
                nownv = plsc.load_gather(nf_v, [nidx])
                qv = jnp.maximum(nownv + nmv - 2.0 * cvec, 0.0)
                qc = jnp.maximum(qv, 1e-30)
                ii = _MAGIC - (plsc.bitcast(qc, jnp.int32) >> 1)
                y = plsc.bitcast(ii, jnp.float32)
                for _ in range(3):
                    y = y * (1.5 - 0.5 * qc * y * y)
                dvv = qv * y
                ev = jnp.exp(-dvv)
                svv = jnp.where(mask8, s1v, s2v)
                numt = _sumall(ev * (svv - avec))
                zv = _sumall(ev) + _K8E
                r = numt / zv
                outv = 1.0 / (1.0 + jnp.exp(-r))
                pos = jnp.full((_L,), c * _CHUNK + e, jnp.int32)
                plsc.store_scatter(out_v, [pos], outv, mask=lane0)

            def ebody(h, ecarry):
                edge_compute(h)
                return ecarry

            lax.fori_loop(0, 16, ebody, 0)

        d = idx_desc(0, 0)
        d.start()
        d.wait()
        for g in gather_descs(0, 0, 0):
            g.start()

        def g2_body(g2, carry):
            for gp in (0, 1):
                g = 2 * g2 + gp
                for j in range(_GRP):
                    c = g * _GRP + j
                    sp = j % 2
                    if j == 0:
                        @pl.when((g + 1) * _GRP < nch_t)
                        def _(g=g, gp=gp):
                            idx_desc(1 - gp, g + 1).start()
                    if j == _GRP - 1:
                        @pl.when(c + 1 < nch_t)
                        def _(g=g, gp=gp, sp=sp):
                            idx_desc(1 - gp, g + 1).wait()
                            for gg in gather_descs(1 - sp, 1 - gp, 0):
                                gg.start()
                    else:
                        @pl.when(c + 1 < nch_t)
                        def _(gp=gp, sp=sp, j=j):
                            for gg in gather_descs(1 - sp, gp, j + 1):
                                gg.start()

                    @pl.when(c < nch_t)
                    def _(c=c, sp=sp, gp=gp, j=j):
                        for gg in gather_descs(sp, gp, j):
                            gg.wait()
                        compute_chunk(c, sp, gp, j)
            return carry

        lax.fori_loop(0, ng2, g2_body, 0)

        ebase = base * _CHUNK

        @pl.when(is_hi)
        def _():
            pltpu.sync_copy(out_v,
                            out_hbm.at[pl.ds(ebase, nch_hi * _CHUNK)])

        @pl.when(jnp.logical_not(is_hi))
        def _():
            pltpu.sync_copy(out_v.at[pl.ds(0, nch_lo * _CHUNK)],
                            out_hbm.at[pl.ds(ebase, nch_lo * _CHUNK)])

    return k(cidx, t_tab, u_tab, nf_tab)


def kernel(src, dst, feats, f_w, f_b, g_w, g_b):
    n_edges = src.shape[0]
    n_nodes = feats.shape[0]
    src = src.astype(jnp.int32)
    dst = dst.astype(jnp.int32)

    rk = jax.random.key(42)
    mid0 = jax.random.randint(jax.random.fold_in(rk, 0), (n_edges, _NS), 0,
                              n_nodes).astype(jnp.int32)
    mid1 = jax.random.randint(jax.random.fold_in(rk, 1), (n_edges, _NS), 0,
                              n_nodes).astype(jnp.int32)

    t_tab, u_tab, nf_tab = _tables(feats, f_w, f_b, g_w, g_b)

    nch = n_edges // _CHUNK
    srcr = src.reshape(nch, _CHUNK)
    dstr = dst.reshape(nch, _CHUNK)
    m0 = mid0.reshape(nch, _CHUNK, _NS)
    m1 = mid1.reshape(nch, _CHUNK, _NS)
    midc = jnp.concatenate([m0, m1], axis=2).reshape(nch, 256)
    cidx = jnp.concatenate([srcr, dstr, dstr, srcr, midc], axis=1)

    return _sc_edges(cidx, t_tab, u_tab, nf_tab, n_edges)
